# baseline (speedup 1.0000x reference)
import functools

import numpy as np
import jax
import jax.numpy as jnp
from jax import lax
from jax.experimental import pallas as pl
from jax.experimental.pallas import tpu as pltpu
from jax.experimental.pallas import tpu_sc as plsc

H, W = 640, 474
HW = H * W
B = 8
M = 200000
NC, NS = 2, 16
CHUNK = 128
NCH = 208
MP2 = NS * NCH * CHUNK
PW = 2 * HW
TSLICE = PW // NS
HSL = TSLICE // 2


def _sc_body(idx_hbm, val_hbm, o_hbm, sh, idx_v, val_v, zb, sem):
    c = lax.axis_index("c")
    s = lax.axis_index("s")
    z0 = s * TSLICE

    def zloop(i, carry):
        zb[pl.ds(i * 16, 16)] = jnp.zeros((16,), jnp.float32)
        return carry
    lax.fori_loop(0, HSL // 16, zloop, 0)

    for p in range(4):
        pltpu.sync_copy(zb, sh.at[pl.ds(z0, HSL)])
        pltpu.sync_copy(zb, sh.at[pl.ds(z0 + HSL, HSL)])
        b = c * 4 + p
        pltpu.sync_copy(idx_hbm.at[b, s], idx_v)
        pltpu.sync_copy(val_hbm.at[b, s], val_v)
        plsc.subcore_barrier()

        GK = 8

        def sloop(g, carry):
            for k in range(GK):
                j = g * GK + k
                pltpu.async_copy(val_v.at[j], sh.at[idx_v.at[j]], sem,
                                 add=True)

            @pl.when(g > 0)
            def _():
                gp = (g - 1) * GK
                pltpu.make_async_copy(val_hbm.at[b, s, pl.ds(gp, GK)],
                                      val_v.at[pl.ds(gp, GK)], sem).wait()
            return carry
        lax.fori_loop(0, NCH // GK, sloop, 0)
        gl = NCH - GK
        pltpu.make_async_copy(val_hbm.at[b, s, pl.ds(gl, GK)],
                              val_v.at[pl.ds(gl, GK)], sem).wait()
        plsc.subcore_barrier()

        base = b * PW
        pltpu.sync_copy(sh.at[pl.ds(z0, HSL)], zb)
        pltpu.sync_copy(zb, o_hbm.at[pl.ds(base + z0, HSL)])
        pltpu.sync_copy(sh.at[pl.ds(z0 + HSL, HSL)], zb)
        pltpu.sync_copy(zb, o_hbm.at[pl.ds(base + z0 + HSL, HSL)])
        plsc.subcore_barrier()
        lax.fori_loop(0, HSL // 16, zloop, 0)


@functools.cache
def _sc_scatter():
    return pl.kernel(
        _sc_body,
        out_type=jax.ShapeDtypeStruct((B * PW,), jnp.float32),
        mesh=plsc.VectorSubcoreMesh(core_axis_name="c", subcore_axis_name="s",
                                    num_cores=NC, num_subcores=NS),
        scratch_types=[
            pltpu.VMEM_SHARED((PW,), jnp.float32),
            pltpu.VMEM((NCH, CHUNK), jnp.int32),
            pltpu.VMEM((NCH, CHUNK), jnp.float32),
            pltpu.VMEM((HSL,), jnp.float32),
            pltpu.SemaphoreType.DMA,
        ],
    )


def _dft_body(g_ref, cr_ref, ci_ref, ar_ref, ai_ref, o_ref):
    f32 = jnp.float32
    g = g_ref[0].astype(jnp.bfloat16)
    cr = cr_ref[...]
    ci = ci_ref[...]
    ar = ar_ref[...]
    ai = ai_ref[...]
    t_r = jnp.dot(g, cr, preferred_element_type=f32)
    t_i = jnp.dot(g, ci, preferred_element_type=f32)
    t_r16 = t_r.astype(jnp.bfloat16)
    t_i16 = t_i.astype(jnp.bfloat16)
    i_r = (jnp.dot(ar, t_r16, preferred_element_type=f32)
           - jnp.dot(ai, t_i16, preferred_element_type=f32))
    i_i = (jnp.dot(ar, t_i16, preferred_element_type=f32)
           + jnp.dot(ai, t_r16, preferred_element_type=f32))
    o_ref[0] = jnp.sqrt(i_r * i_r + i_i * i_i)


def _dft_mats():
    nw = np.arange(W, dtype=np.int64)
    tw = (2.0 * np.pi / W) * ((nw[:, None] * nw[None, :]) % W)
    br = np.cos(tw) / np.sqrt(W)
    bi = np.sin(tw) / np.sqrt(W)
    cr = np.zeros((2 * W, W))
    ci = np.zeros((2 * W, W))
    cr[0::2] = br
    cr[1::2] = -bi
    ci[0::2] = bi
    ci[1::2] = br
    nh = np.arange(H, dtype=np.int64)
    th = (2.0 * np.pi / H) * ((nh[:, None] * nh[None, :]) % H)
    ar = np.cos(th) / np.sqrt(H)
    ai = np.sin(th) / np.sqrt(H)
    bf16 = jnp.bfloat16
    return (jnp.asarray(cr, bf16), jnp.asarray(ci, bf16),
            jnp.asarray(ar, bf16), jnp.asarray(ai, bf16))


_dft = pl.pallas_call(
    _dft_body,
    grid=(B,),
    in_specs=[
        pl.BlockSpec((1, H, 2 * W), lambda b: (b, 0, 0)),
        pl.BlockSpec((2 * W, W), lambda b: (0, 0)),
        pl.BlockSpec((2 * W, W), lambda b: (0, 0)),
        pl.BlockSpec((H, H), lambda b: (0, 0)),
        pl.BlockSpec((H, H), lambda b: (0, 0)),
    ],
    out_specs=pl.BlockSpec((1, H, W), lambda b: (b, 0, 0)),
    out_shape=jax.ShapeDtypeStruct((B, H, W), jnp.float32),
)


def kernel(kspace_real, kspace_imag, ktraj, dcomp):
    tr = ktraj
    gx = jnp.mod(jnp.floor((tr[:, 0, :] + np.pi) / (2.0 * np.pi) * H),
                 H).astype(jnp.int32)
    gy = jnp.mod(jnp.floor((tr[:, 1, :] + np.pi) / (2.0 * np.pi) * W),
                 W).astype(jnp.int32)
    sx = jnp.mod(gx + H // 2, H)
    sy = jnp.mod(gy + W // 2, W)
    sign = (1 - 2 * jnp.bitwise_and(sx + sy, 1)).astype(jnp.float32)
    wgt = dcomp * sign
    vr = kspace_real[:, 0, :] * wgt
    vi = kspace_imag[:, 0, :] * wgt
    iw = 2 * (sx * W + sy)
    idx2 = jnp.stack([iw, iw + 1], axis=-1).reshape(B, 2 * M)
    val2 = jnp.stack([vr, vi], axis=-1).reshape(B, 2 * M)

    pad = ((0, 0), (0, MP2 - 2 * M))
    idxp = jnp.pad(idx2, pad).reshape(B, NS, NCH, CHUNK)
    valp = jnp.pad(val2, pad).reshape(B, NS, NCH, CHUNK)

    gil = _sc_scatter()(idxp, valp)
    cr, ci, ar, ai = _dft_mats()
    mag = _dft(gil.reshape(B, H, 2 * W), cr, ci, ar, ai)
    return mag[..., None]


# speedup vs baseline: 2.6861x; 2.6861x over previous
import functools

import numpy as np
import jax
import jax.numpy as jnp
from jax import lax
from jax.experimental import pallas as pl
from jax.experimental.pallas import tpu as pltpu
from jax.experimental.pallas import tpu_sc as plsc

H, W = 640, 474
HW = H * W
B = 8
M = 200000
NC, NS = 2, 16
CHUNK = 128
NCH = 104
NROWS = 98
MP = NS * NCH * CHUNK
SH_WORDS = HW
TSLICE = SH_WORDS // NS


def _sc_body(idx_hbm, vr_hbm, vi_hbm, ore_hbm, oim_hbm,
             sh_re, sh_im, idx_v, vr_v, vi_v, zb, sem_re, sem_im):
    c = lax.axis_index("c")
    s = lax.axis_index("s")
    s0 = s * TSLICE

    def zloop(i, carry):
        zb[pl.ds(i * 16, 16)] = jnp.zeros((16,), jnp.float32)
        return carry

    if True:
        lax.fori_loop(0, TSLICE // 16, zloop, 0)
        pltpu.sync_copy(zb, sh_re.at[pl.ds(s0, TSLICE)])
        pltpu.sync_copy(zb, sh_im.at[pl.ds(s0, TSLICE)])
        b = c
        pltpu.sync_copy(idx_hbm.at[b, s], idx_v)
        pltpu.sync_copy(vr_hbm.at[b, s], vr_v)
        pltpu.sync_copy(vi_hbm.at[b, s], vi_v)
        plsc.subcore_barrier()

        GK = 8
        NG = NROWS // GK

        def sloop(g, carry):
            for k in range(GK):
                j = g * GK + k
                pltpu.async_copy(vr_v.at[j], sh_re.at[idx_v.at[j]], sem_re,
                                 add=True)
                pltpu.async_copy(vi_v.at[j], sh_im.at[idx_v.at[j]], sem_im,
                                 add=True)

            @pl.when(g > 0)
            def _():
                gp = (g - 1) * GK
                pltpu.make_async_copy(vr_hbm.at[b, s, pl.ds(gp, GK)],
                                      vr_v.at[pl.ds(gp, GK)], sem_re).wait()
                pltpu.make_async_copy(vi_hbm.at[b, s, pl.ds(gp, GK)],
                                      vi_v.at[pl.ds(gp, GK)], sem_im).wait()
            return carry
        lax.fori_loop(0, NCH // GK, sloop, 0)
        gl = NCH - GK
        pltpu.make_async_copy(vr_hbm.at[b, s, pl.ds(gl, GK)],
                              vr_v.at[pl.ds(gl, GK)], sem_re).wait()
        pltpu.make_async_copy(vi_hbm.at[b, s, pl.ds(gl, GK)],
                              vi_v.at[pl.ds(gl, GK)], sem_im).wait()
        plsc.subcore_barrier()

        base = b * HW
        pltpu.sync_copy(sh_re.at[pl.ds(s0, TSLICE)], zb)
        pltpu.sync_copy(zb, ore_hbm.at[pl.ds(base + s0, TSLICE)])
        pltpu.sync_copy(sh_im.at[pl.ds(s0, TSLICE)], zb)
        pltpu.sync_copy(zb, oim_hbm.at[pl.ds(base + s0, TSLICE)])
        plsc.subcore_barrier()


@functools.cache
def _sc_scatter():
    return pl.kernel(
        _sc_body,
        out_type=(jax.ShapeDtypeStruct((2 * HW,), jnp.float32),
                  jax.ShapeDtypeStruct((2 * HW,), jnp.float32)),
        mesh=plsc.VectorSubcoreMesh(core_axis_name="c", subcore_axis_name="s",
                                    num_cores=NC, num_subcores=NS),
        scratch_types=[
            pltpu.VMEM_SHARED((SH_WORDS,), jnp.float32),
            pltpu.VMEM_SHARED((SH_WORDS,), jnp.float32),
            pltpu.VMEM((NCH, CHUNK), jnp.int32),
            pltpu.VMEM((NCH, CHUNK), jnp.float32),
            pltpu.VMEM((NCH, CHUNK), jnp.float32),
            pltpu.VMEM((TSLICE,), jnp.float32),
            pltpu.SemaphoreType.DMA,
            pltpu.SemaphoreType.DMA,
        ],
    )


def _dft_body(gr_ref, gi_ref, ar_ref, ai_ref, br_ref, bi_ref, o_ref):
    f32 = jnp.float32
    bf16 = jnp.bfloat16
    gr = gr_ref[0].astype(bf16)
    gi = gi_ref[0].astype(bf16)
    ar = ar_ref[...]
    ai = ai_ref[...]
    br = br_ref[...]
    bi = bi_ref[...]
    t_r = (jnp.dot(gr, br, preferred_element_type=f32)
           - jnp.dot(gi, bi, preferred_element_type=f32))
    t_i = (jnp.dot(gr, bi, preferred_element_type=f32)
           + jnp.dot(gi, br, preferred_element_type=f32))
    t_r16 = t_r.astype(bf16)
    t_i16 = t_i.astype(bf16)
    i_r = (jnp.dot(ar, t_r16, preferred_element_type=f32)
           - jnp.dot(ai, t_i16, preferred_element_type=f32))
    i_i = (jnp.dot(ar, t_i16, preferred_element_type=f32)
           + jnp.dot(ai, t_r16, preferred_element_type=f32))
    o_ref[0] = jnp.sqrt(i_r * i_r + i_i * i_i)


def _dft_mats():
    nh = np.arange(H, dtype=np.int64)
    th = (2.0 * np.pi / H) * ((nh[:, None] * nh[None, :]) % H)
    ar = jnp.asarray(np.cos(th) / np.sqrt(H), jnp.bfloat16)
    ai = jnp.asarray(np.sin(th) / np.sqrt(H), jnp.bfloat16)
    nw = np.arange(W, dtype=np.int64)
    tw = (2.0 * np.pi / W) * ((nw[:, None] * nw[None, :]) % W)
    br = jnp.asarray(np.cos(tw) / np.sqrt(W), jnp.bfloat16)
    bi = jnp.asarray(np.sin(tw) / np.sqrt(W), jnp.bfloat16)
    return ar, ai, br, bi


_dft = pl.pallas_call(
    _dft_body,
    grid=(2,),
    in_specs=[
        pl.BlockSpec((1, H, W), lambda b: (b, 0, 0)),
        pl.BlockSpec((1, H, W), lambda b: (b, 0, 0)),
        pl.BlockSpec((H, H), lambda b: (0, 0)),
        pl.BlockSpec((H, H), lambda b: (0, 0)),
        pl.BlockSpec((W, W), lambda b: (0, 0)),
        pl.BlockSpec((W, W), lambda b: (0, 0)),
    ],
    out_specs=pl.BlockSpec((1, H, W), lambda b: (b, 0, 0)),
    out_shape=jax.ShapeDtypeStruct((2, H, W), jnp.float32),
)


def kernel(kspace_real, kspace_imag, ktraj, dcomp):
    tr = ktraj
    gx = jnp.mod(jnp.floor((tr[:, 0, :] + np.pi) / (2.0 * np.pi) * H),
                 H).astype(jnp.int32)
    gy = jnp.mod(jnp.floor((tr[:, 1, :] + np.pi) / (2.0 * np.pi) * W),
                 W).astype(jnp.int32)
    sx = jnp.mod(gx + H // 2, H)
    sy = jnp.mod(gy + W // 2, W)
    sign = (1 - 2 * jnp.bitwise_and(sx + sy, 1)).astype(jnp.float32)
    wgt = dcomp * sign
    vr = kspace_real[:, 0, :] * wgt
    vi = kspace_imag[:, 0, :] * wgt
    idx = sx * W + sy

    pad = ((0, 0), (0, MP - M))
    idxp = jnp.pad(idx, pad).reshape(B, NS, NCH, CHUNK)
    vrp = jnp.pad(vr, pad).reshape(B, NS, NCH, CHUNK)
    vip = jnp.pad(vi, pad).reshape(B, NS, NCH, CHUNK)

    ar, ai, br, bi = _dft_mats()
    mags = []
    for p in range(4):
        sl = slice(2 * p, 2 * p + 2)
        gre, gim = _sc_scatter()(idxp[sl], vrp[sl], vip[sl])
        mags.append(_dft(gre.reshape(2, H, W), gim.reshape(2, H, W),
                         ar, ai, br, bi))
    return jnp.concatenate(mags, axis=0)[..., None]


# speedup vs baseline: 2.9420x; 1.0953x over previous
import functools

import numpy as np
import jax
import jax.numpy as jnp
from jax import lax
from jax.experimental import pallas as pl
from jax.experimental.pallas import tpu as pltpu
from jax.experimental.pallas import tpu_sc as plsc

H, W = 640, 474
HW = H * W
B = 8
M = 200000
NC, NS = 2, 16
CHUNK = 128
NCH = 104
NROWS = 98
MP = NS * NCH * CHUNK
SH_WORDS = HW
TSLICE = SH_WORDS // NS


def _sc_body(idx_hbm, pk_hbm, ore_hbm, oim_hbm,
             sh_re, sh_im, idx_v, pk_v, vr_v, vi_v, zb, sem_re, sem_im):
    c = lax.axis_index("c")
    s = lax.axis_index("s")
    s0 = s * TSLICE

    def zloop(i, carry):
        zb[pl.ds(i * 16, 16)] = jnp.zeros((16,), jnp.float32)
        return carry

    for p in range(4):
        lax.fori_loop(0, TSLICE // 16, zloop, 0)
        pltpu.sync_copy(zb, sh_re.at[pl.ds(s0, TSLICE)])
        pltpu.sync_copy(zb, sh_im.at[pl.ds(s0, TSLICE)])
        b = c * 4 + p
        pltpu.sync_copy(idx_hbm.at[b, s], idx_v)
        pltpu.sync_copy(pk_hbm.at[b, s], pk_v)
        plsc.subcore_barrier()

        GK = 8

        def sloop(g, carry):
            for k in range(GK):
                j = g * GK + k
                pk_row = pk_v.at[j]
                vr_row = vr_v.at[j]
                vi_row = vi_v.at[j]
                for q in range(CHUNK // 16):
                    w = pk_row[pl.ds(q * 16, 16)]
                    re = lax.bitcast_convert_type(w << 16, jnp.float32)
                    im = lax.bitcast_convert_type(w & jnp.int32(-65536),
                                                  jnp.float32)
                    vr_row[pl.ds(q * 16, 16)] = re
                    vi_row[pl.ds(q * 16, 16)] = im
                pltpu.async_copy(vr_row, sh_re.at[idx_v.at[j]], sem_re,
                                 add=True)
                pltpu.async_copy(vi_row, sh_im.at[idx_v.at[j]], sem_im,
                                 add=True)

            @pl.when(g > 0)
            def _():
                gp = (g - 1) * GK
                pltpu.make_async_copy(pk_hbm.at[b, s, pl.ds(gp, GK)],
                                      pk_v.at[pl.ds(gp, GK)], sem_re).wait()
                pltpu.make_async_copy(pk_hbm.at[b, s, pl.ds(gp, GK)],
                                      pk_v.at[pl.ds(gp, GK)], sem_im).wait()
            return carry
        lax.fori_loop(0, NCH // GK, sloop, 0)
        gl = NCH - GK
        pltpu.make_async_copy(pk_hbm.at[b, s, pl.ds(gl, GK)],
                              pk_v.at[pl.ds(gl, GK)], sem_re).wait()
        pltpu.make_async_copy(pk_hbm.at[b, s, pl.ds(gl, GK)],
                              pk_v.at[pl.ds(gl, GK)], sem_im).wait()
        plsc.subcore_barrier()

        base = b * HW
        pltpu.sync_copy(sh_re.at[pl.ds(s0, TSLICE)], zb)
        pltpu.sync_copy(zb, ore_hbm.at[pl.ds(base + s0, TSLICE)])
        pltpu.sync_copy(sh_im.at[pl.ds(s0, TSLICE)], zb)
        pltpu.sync_copy(zb, oim_hbm.at[pl.ds(base + s0, TSLICE)])
        plsc.subcore_barrier()


@functools.cache
def _sc_scatter():
    return pl.kernel(
        _sc_body,
        out_type=(jax.ShapeDtypeStruct((B * HW,), jnp.float32),
                  jax.ShapeDtypeStruct((B * HW,), jnp.float32)),
        mesh=plsc.VectorSubcoreMesh(core_axis_name="c", subcore_axis_name="s",
                                    num_cores=NC, num_subcores=NS),
        scratch_types=[
            pltpu.VMEM_SHARED((SH_WORDS,), jnp.float32),
            pltpu.VMEM_SHARED((SH_WORDS,), jnp.float32),
            pltpu.VMEM((NCH, CHUNK), jnp.int32),
            pltpu.VMEM((NCH, CHUNK), jnp.int32),
            pltpu.VMEM((NCH, CHUNK), jnp.float32),
            pltpu.VMEM((NCH, CHUNK), jnp.float32),
            pltpu.VMEM((TSLICE,), jnp.float32),
            pltpu.SemaphoreType.DMA,
            pltpu.SemaphoreType.DMA,
        ],
    )


def _dft_body(gr_ref, gi_ref, ar_ref, ai_ref, br_ref, bi_ref, o_ref):
    f32 = jnp.float32
    bf16 = jnp.bfloat16
    gr = gr_ref[0].astype(bf16)
    gi = gi_ref[0].astype(bf16)
    ar = ar_ref[...]
    ai = ai_ref[...]
    br = br_ref[...]
    bi = bi_ref[...]
    t_r = (jnp.dot(gr, br, preferred_element_type=f32)
           - jnp.dot(gi, bi, preferred_element_type=f32))
    t_i = (jnp.dot(gr, bi, preferred_element_type=f32)
           + jnp.dot(gi, br, preferred_element_type=f32))
    t_r16 = t_r.astype(bf16)
    t_i16 = t_i.astype(bf16)
    i_r = (jnp.dot(ar, t_r16, preferred_element_type=f32)
           - jnp.dot(ai, t_i16, preferred_element_type=f32))
    i_i = (jnp.dot(ar, t_i16, preferred_element_type=f32)
           + jnp.dot(ai, t_r16, preferred_element_type=f32))
    o_ref[0] = jnp.sqrt(i_r * i_r + i_i * i_i)


def _dft_mats():
    nh = np.arange(H, dtype=np.int64)
    th = (2.0 * np.pi / H) * ((nh[:, None] * nh[None, :]) % H)
    ar = jnp.asarray(np.cos(th) / np.sqrt(H), jnp.bfloat16)
    ai = jnp.asarray(np.sin(th) / np.sqrt(H), jnp.bfloat16)
    nw = np.arange(W, dtype=np.int64)
    tw = (2.0 * np.pi / W) * ((nw[:, None] * nw[None, :]) % W)
    br = jnp.asarray(np.cos(tw) / np.sqrt(W), jnp.bfloat16)
    bi = jnp.asarray(np.sin(tw) / np.sqrt(W), jnp.bfloat16)
    return ar, ai, br, bi


_dft = pl.pallas_call(
    _dft_body,
    grid=(B,),
    in_specs=[
        pl.BlockSpec((1, H, W), lambda b: (b, 0, 0)),
        pl.BlockSpec((1, H, W), lambda b: (b, 0, 0)),
        pl.BlockSpec((H, H), lambda b: (0, 0)),
        pl.BlockSpec((H, H), lambda b: (0, 0)),
        pl.BlockSpec((W, W), lambda b: (0, 0)),
        pl.BlockSpec((W, W), lambda b: (0, 0)),
    ],
    out_specs=pl.BlockSpec((1, H, W), lambda b: (b, 0, 0)),
    out_shape=jax.ShapeDtypeStruct((B, H, W), jnp.float32),
)


def kernel(kspace_real, kspace_imag, ktraj, dcomp):
    tr = ktraj
    gx = jnp.mod(jnp.floor((tr[:, 0, :] + np.pi) / (2.0 * np.pi) * H),
                 H).astype(jnp.int32)
    gy = jnp.mod(jnp.floor((tr[:, 1, :] + np.pi) / (2.0 * np.pi) * W),
                 W).astype(jnp.int32)
    sx = jnp.mod(gx + H // 2, H)
    sy = jnp.mod(gy + W // 2, W)
    sign = (1 - 2 * jnp.bitwise_and(sx + sy, 1)).astype(jnp.float32)
    wgt = dcomp * sign
    vr = kspace_real[:, 0, :] * wgt
    vi = kspace_imag[:, 0, :] * wgt
    idx = sx * W + sy

    pad = ((0, 0), (0, MP - M))
    idxp = jnp.pad(idx, pad).reshape(B, NS, NCH, CHUNK)
    pk = lax.bitcast_convert_type(
        jnp.stack([vr.astype(jnp.bfloat16), vi.astype(jnp.bfloat16)],
                  axis=-1), jnp.int32)
    pkp = jnp.pad(pk, pad).reshape(B, NS, NCH, CHUNK)

    gre, gim = _sc_scatter()(idxp, pkp)
    ar, ai, br, bi = _dft_mats()
    mag = _dft(gre.reshape(B, H, W), gim.reshape(B, H, W), ar, ai, br, bi)
    return mag[..., None]


# speedup vs baseline: 3.1057x; 1.0556x over previous
import functools

import numpy as np
import jax
import jax.numpy as jnp
from jax import lax
from jax.experimental import pallas as pl
from jax.experimental.pallas import tpu as pltpu
from jax.experimental.pallas import tpu_sc as plsc

H, W = 640, 474
HW = H * W
B = 8
M = 200000
NC, NS = 2, 16
CHUNK = 128
NCH = 104
NROWS = 98
MP = NS * NCH * CHUNK
SH_WORDS = HW
TSLICE = SH_WORDS // NS


def _sc_body(idx_hbm, pk_hbm, ore_hbm, oim_hbm,
             sh_re, sh_im, idx_v, pk_v, vr_v, vi_v, zb, stg, sem_re, sem_im):
    c = lax.axis_index("c")
    s = lax.axis_index("s")
    s0 = s * TSLICE

    def zloop(i, carry):
        zb[pl.ds(i * 16, 16)] = jnp.zeros((16,), jnp.float32)
        return carry
    lax.fori_loop(0, TSLICE // 16, zloop, 0)

    for p in range(4):
        pltpu.sync_copy(zb, sh_re.at[pl.ds(s0, TSLICE)])
        pltpu.sync_copy(zb, sh_im.at[pl.ds(s0, TSLICE)])
        b = c * 4 + p
        pltpu.sync_copy(idx_hbm.at[b, s], idx_v)
        pltpu.sync_copy(pk_hbm.at[b, s], pk_v)
        plsc.subcore_barrier()

        GK = 8

        def sloop(g, carry):
            for k in range(GK):
                j = g * GK + k
                pk_row = pk_v.at[j]
                vr_row = vr_v.at[j]
                vi_row = vi_v.at[j]
                for q in range(CHUNK // 16):
                    w = pk_row[pl.ds(q * 16, 16)]
                    re = lax.bitcast_convert_type(w << 16, jnp.float32)
                    im = lax.bitcast_convert_type(w & jnp.int32(-65536),
                                                  jnp.float32)
                    vr_row[pl.ds(q * 16, 16)] = re
                    vi_row[pl.ds(q * 16, 16)] = im
                pltpu.async_copy(vr_row, sh_re.at[idx_v.at[j]], sem_re,
                                 add=True)
                pltpu.async_copy(vi_row, sh_im.at[idx_v.at[j]], sem_im,
                                 add=True)

            @pl.when(g > 0)
            def _():
                gp = (g - 1) * GK
                pltpu.make_async_copy(pk_hbm.at[b, s, pl.ds(gp, GK)],
                                      pk_v.at[pl.ds(gp, GK)], sem_re).wait()
                pltpu.make_async_copy(pk_hbm.at[b, s, pl.ds(gp, GK)],
                                      pk_v.at[pl.ds(gp, GK)], sem_im).wait()
            return carry
        lax.fori_loop(0, NCH // GK, sloop, 0)
        gl = NCH - GK
        pltpu.make_async_copy(pk_hbm.at[b, s, pl.ds(gl, GK)],
                              pk_v.at[pl.ds(gl, GK)], sem_re).wait()
        pltpu.make_async_copy(pk_hbm.at[b, s, pl.ds(gl, GK)],
                              pk_v.at[pl.ds(gl, GK)], sem_im).wait()
        plsc.subcore_barrier()

        base = b * HW
        pltpu.sync_copy(sh_re.at[pl.ds(s0, TSLICE)], stg)
        pltpu.sync_copy(stg, ore_hbm.at[pl.ds(base + s0, TSLICE)])
        pltpu.sync_copy(sh_im.at[pl.ds(s0, TSLICE)], stg)
        pltpu.sync_copy(stg, oim_hbm.at[pl.ds(base + s0, TSLICE)])
        plsc.subcore_barrier()


@functools.cache
def _sc_scatter():
    return pl.kernel(
        _sc_body,
        out_type=(jax.ShapeDtypeStruct((B * HW,), jnp.float32),
                  jax.ShapeDtypeStruct((B * HW,), jnp.float32)),
        mesh=plsc.VectorSubcoreMesh(core_axis_name="c", subcore_axis_name="s",
                                    num_cores=NC, num_subcores=NS),
        scratch_types=[
            pltpu.VMEM_SHARED((SH_WORDS,), jnp.float32),
            pltpu.VMEM_SHARED((SH_WORDS,), jnp.float32),
            pltpu.VMEM((NCH, CHUNK), jnp.int32),
            pltpu.VMEM((NCH, CHUNK), jnp.int32),
            pltpu.VMEM((NCH, CHUNK), jnp.float32),
            pltpu.VMEM((NCH, CHUNK), jnp.float32),
            pltpu.VMEM((TSLICE,), jnp.float32),
            pltpu.VMEM((TSLICE,), jnp.float32),
            pltpu.SemaphoreType.DMA,
            pltpu.SemaphoreType.DMA,
        ],
    )


def _dft_body(gr_ref, gi_ref, ar_ref, ai_ref, br_ref, bi_ref, o_ref):
    f32 = jnp.float32
    bf16 = jnp.bfloat16
    gr = gr_ref[0].astype(bf16)
    gi = gi_ref[0].astype(bf16)
    ar = ar_ref[...]
    ai = ai_ref[...]
    br = br_ref[...]
    bi = bi_ref[...]
    t_r = (jnp.dot(gr, br, preferred_element_type=f32)
           - jnp.dot(gi, bi, preferred_element_type=f32))
    t_i = (jnp.dot(gr, bi, preferred_element_type=f32)
           + jnp.dot(gi, br, preferred_element_type=f32))
    t_r16 = t_r.astype(bf16)
    t_i16 = t_i.astype(bf16)
    i_r = (jnp.dot(ar, t_r16, preferred_element_type=f32)
           - jnp.dot(ai, t_i16, preferred_element_type=f32))
    i_i = (jnp.dot(ar, t_i16, preferred_element_type=f32)
           + jnp.dot(ai, t_r16, preferred_element_type=f32))
    o_ref[0] = jnp.sqrt(i_r * i_r + i_i * i_i)


def _dft_mats():
    nh = np.arange(H, dtype=np.int64)
    th = (2.0 * np.pi / H) * ((nh[:, None] * nh[None, :]) % H)
    ar = jnp.asarray(np.cos(th) / np.sqrt(H), jnp.bfloat16)
    ai = jnp.asarray(np.sin(th) / np.sqrt(H), jnp.bfloat16)
    nw = np.arange(W, dtype=np.int64)
    tw = (2.0 * np.pi / W) * ((nw[:, None] * nw[None, :]) % W)
    br = jnp.asarray(np.cos(tw) / np.sqrt(W), jnp.bfloat16)
    bi = jnp.asarray(np.sin(tw) / np.sqrt(W), jnp.bfloat16)
    return ar, ai, br, bi


_dft = pl.pallas_call(
    _dft_body,
    grid=(B,),
    in_specs=[
        pl.BlockSpec((1, H, W), lambda b: (b, 0, 0)),
        pl.BlockSpec((1, H, W), lambda b: (b, 0, 0)),
        pl.BlockSpec((H, H), lambda b: (0, 0)),
        pl.BlockSpec((H, H), lambda b: (0, 0)),
        pl.BlockSpec((W, W), lambda b: (0, 0)),
        pl.BlockSpec((W, W), lambda b: (0, 0)),
    ],
    out_specs=pl.BlockSpec((1, H, W), lambda b: (b, 0, 0)),
    out_shape=jax.ShapeDtypeStruct((B, H, W), jnp.float32),
)


def kernel(kspace_real, kspace_imag, ktraj, dcomp):
    tr = ktraj
    gx = jnp.mod(jnp.floor((tr[:, 0, :] + np.pi) / (2.0 * np.pi) * H),
                 H).astype(jnp.int32)
    gy = jnp.mod(jnp.floor((tr[:, 1, :] + np.pi) / (2.0 * np.pi) * W),
                 W).astype(jnp.int32)
    sx = jnp.mod(gx + H // 2, H)
    sy = jnp.mod(gy + W // 2, W)
    sign = (1 - 2 * jnp.bitwise_and(sx + sy, 1)).astype(jnp.float32)
    wgt = dcomp * sign
    vr = kspace_real[:, 0, :] * wgt
    vi = kspace_imag[:, 0, :] * wgt
    idx = sx * W + sy

    pad = ((0, 0), (0, MP - M))
    idxp = jnp.pad(idx, pad).reshape(B, NS, NCH, CHUNK)
    pk = lax.bitcast_convert_type(
        jnp.stack([vr.astype(jnp.bfloat16), vi.astype(jnp.bfloat16)],
                  axis=-1), jnp.int32)
    pkp = jnp.pad(pk, pad).reshape(B, NS, NCH, CHUNK)

    gre, gim = _sc_scatter()(idxp, pkp)
    ar, ai, br, bi = _dft_mats()
    mag = _dft(gre.reshape(B, H, W), gim.reshape(B, H, W), ar, ai, br, bi)
    return mag[..., None]
